# baseline (speedup 1.0000x reference)
import functools

import jax
import jax.numpy as jnp
from jax import lax
from jax.experimental import pallas as pl
from jax.experimental.pallas import tpu as pltpu
from jax.experimental.pallas import tpu_sc as plsc

N = 10000
E = 320000
D = 128
T = 4
NSTEPS = 8
NGRAPHS = 16
HID = 256

NCORES = 2
NSUB = 16
NWORKERS = NCORES * NSUB
CHUNK = 128
NBUF = 2
CPW = 80
ROUNDS = CPW // NBUF
EPAD = NWORKERS * CPW * CHUNK
ROWS_PT = 640
ACC_ROWS = NSUB * ROWS_PT
LAST_ROWS = N - 15 * ROWS_PT


@functools.lru_cache(maxsize=None)
def _make_sc_edge():
    @functools.partial(
        pl.kernel,
        mesh=plsc.VectorSubcoreMesh(core_axis_name="c", subcore_axis_name="s"),
        out_type=jax.ShapeDtypeStruct((NCORES, N, D), jnp.float32),
        scratch_types=[
            pltpu.VMEM((CPW, CHUNK), jnp.int32),
            pltpu.VMEM((CPW, CHUNK), jnp.int32),
            pltpu.VMEM((CHUNK, D), jnp.float32),
            pltpu.VMEM_SHARED((ACC_ROWS, D), jnp.float32),
            pltpu.SemaphoreType.DMA,
        ],
    )
    def _sc_edge(table_hbm, gidx_hbm, dst_hbm, zeros_hbm, out_hbm,
                 idx_v, dst_v, rows_v, acc, sem):
        c = lax.axis_index("c")
        s = lax.axis_index("s")
        wid = c * NSUB + s
        pltpu.sync_copy(gidx_hbm.at[wid], idx_v)
        pltpu.sync_copy(dst_hbm.at[wid], dst_v)
        pltpu.sync_copy(zeros_hbm, acc.at[pl.ds(s * ROWS_PT, ROWS_PT)])
        plsc.subcore_barrier()

        def chunk_body(g, carry):
            pltpu.async_copy(table_hbm.at[idx_v.at[g]], rows_v, sem).wait()
            pltpu.sync_copy(rows_v, acc.at[dst_v.at[g]], add=True)
            return carry

        lax.fori_loop(0, CPW, chunk_body, 0)
        plsc.subcore_barrier()

        @pl.when(s < NSUB - 1)
        def _():
            pltpu.sync_copy(acc.at[pl.ds(s * ROWS_PT, ROWS_PT)],
                            out_hbm.at[c, pl.ds(s * ROWS_PT, ROWS_PT)])

        @pl.when(s == NSUB - 1)
        def _():
            pltpu.sync_copy(acc.at[pl.ds((NSUB - 1) * ROWS_PT, LAST_ROWS)],
                            out_hbm.at[c, pl.ds((NSUB - 1) * ROWS_PT, LAST_ROWS)])

    return _sc_edge


RB = 1000


def _wh_body(h_ref, w_ref, b_ref, o_ref):
    o_ref[...] = (jnp.dot(h_ref[...], w_ref[...],
                          preferred_element_type=jnp.float32) + b_ref[...])


def _tc_wh(h, wcat, bcat):
    return pl.pallas_call(
        _wh_body,
        grid=(N // RB,),
        in_specs=[
            pl.BlockSpec((RB, D), lambda i: (i, 0)),
            pl.BlockSpec((D, T * D), lambda i: (0, 0)),
            pl.BlockSpec((1, T * D), lambda i: (0, 0)),
        ],
        out_specs=pl.BlockSpec((RB, T * D), lambda i: (i, 0)),
        out_shape=jax.ShapeDtypeStruct((N, T * D), jnp.float32),
    )(h, wcat, bcat)


def _gru_body(h_ref, a_ref, wih_ref, whh_ref, bih_ref, bhh_ref, o_ref):
    hb = h_ref[...]
    av = a_ref[...].astype(jnp.float32)
    ab = av[0] + av[1]
    gi = jnp.dot(ab, wih_ref[...], preferred_element_type=jnp.float32) + bih_ref[...]
    gh = jnp.dot(hb, whh_ref[...], preferred_element_type=jnp.float32) + bhh_ref[...]
    r = jax.nn.sigmoid(gi[:, :D] + gh[:, :D])
    z = jax.nn.sigmoid(gi[:, D:2 * D] + gh[:, D:2 * D])
    n = jnp.tanh(gi[:, 2 * D:] + r * gh[:, 2 * D:])
    o_ref[...] = (1.0 - z) * n + z * hb


def _tc_gru(h, a2, wiht, whht, bih, bhh):
    return pl.pallas_call(
        _gru_body,
        grid=(N // RB,),
        in_specs=[
            pl.BlockSpec((RB, D), lambda i: (i, 0)),
            pl.BlockSpec((NCORES, RB, D), lambda i: (0, i, 0)),
            pl.BlockSpec((D, 3 * D), lambda i: (0, 0)),
            pl.BlockSpec((D, 3 * D), lambda i: (0, 0)),
            pl.BlockSpec((1, 3 * D), lambda i: (0, 0)),
            pl.BlockSpec((1, 3 * D), lambda i: (0, 0)),
        ],
        out_specs=pl.BlockSpec((RB, D), lambda i: (i, 0)),
        out_shape=jax.ShapeDtypeStruct((N, D), jnp.float32),
    )(h, a2, wiht, whht, bih, bhh)


def _pool_body(h_ref, gid_ref, w1_ref, b1_ref, w2_ref, b2_ref, o_ref):
    hv = h_ref[...]
    gid = gid_ref[...]
    gids = lax.broadcasted_iota(jnp.int32, (1, NGRAPHS), 1)
    onehot = (gid == gids).astype(jnp.float32)
    sums = lax.dot_general(onehot, hv, (((0,), (0,)), ((), ())),
                           preferred_element_type=jnp.float32)
    cnt = jnp.sum(onehot, axis=0)[:, None]
    feat = sums / jnp.maximum(cnt, 1.0)
    hid = jax.nn.relu(jnp.dot(feat, w1_ref[...],
                              preferred_element_type=jnp.float32) + b1_ref[...])
    logits = jnp.dot(hid, w2_ref[...],
                     preferred_element_type=jnp.float32) + b2_ref[...]
    o_ref[...] = jax.nn.sigmoid(logits)


def _tc_pool(h, gid2d, w1, b1, w2, b2):
    return pl.pallas_call(
        _pool_body,
        out_shape=jax.ShapeDtypeStruct((NGRAPHS, 1), jnp.float32),
    )(h, gid2d, w1, b1, w2, b2)


def kernel(x, edge_index, edge_types, graph_ids, We, be, w_ih, w_hh,
           b_ih, b_hh, W1, b1, W2, b2):
    src = edge_index[0].astype(jnp.int32)
    dst = edge_index[1].astype(jnp.int32)
    et = edge_types.astype(jnp.int32)

    gidx = src * T + et
    pad = EPAD - E
    gidx_p = jnp.concatenate([gidx, jnp.zeros((pad,), jnp.int32)])
    dst_p = jnp.concatenate([dst, jnp.full((pad,), N, jnp.int32)])
    gidx_p = gidx_p.reshape(NWORKERS, CPW, CHUNK)
    dst_p = dst_p.reshape(NWORKERS, CPW, CHUNK)
    zeros = jnp.zeros((ROWS_PT, D), jnp.float32)

    wcat = jnp.transpose(We, (2, 0, 1)).reshape(D, T * D)
    bcat = be.reshape(1, T * D)
    wiht = w_ih.T
    whht = w_hh.T
    bih = b_ih.reshape(1, 3 * D)
    bhh = b_hh.reshape(1, 3 * D)

    h = x
    for _ in range(NSTEPS):
        wh = _tc_wh(h, wcat, bcat)
        table = wh.reshape(N * T, D)
        a2 = _make_sc_edge()(table, gidx_p, dst_p, zeros)
        h = _tc_gru(h, a2, wiht, whht, bih, bhh)

    gid2d = graph_ids.astype(jnp.int32).reshape(N, 1)
    return _tc_pool(h, gid2d, W1, b1.reshape(1, HID), W2, b2.reshape(1, 1))


# speedup vs baseline: 1.4771x; 1.4771x over previous
import functools

import jax
import jax.numpy as jnp
from jax import lax
from jax.experimental import pallas as pl
from jax.experimental.pallas import tpu as pltpu
from jax.experimental.pallas import tpu_sc as plsc

N = 10000
E = 320000
D = 128
T = 4
NSTEPS = 8
NGRAPHS = 16
HID = 256

NCORES = 2
NSUB = 16
NWORKERS = NCORES * NSUB
CHUNK = 128
CPW = 79
EPAD = NWORKERS * CPW * CHUNK
ROWS_PT = 632
ACC_ROWS = NSUB * ROWS_PT
LAST_ROWS = N - 15 * ROWS_PT


@functools.lru_cache(maxsize=None)
def _make_sc_edge():
    @functools.partial(
        pl.kernel,
        mesh=plsc.VectorSubcoreMesh(core_axis_name="c", subcore_axis_name="s"),
        out_type=jax.ShapeDtypeStruct((NCORES, N, D), jnp.float32),
        scratch_types=[
            pltpu.VMEM((CPW, CHUNK), jnp.int32),
            pltpu.VMEM((CPW, CHUNK), jnp.int32),
            pltpu.VMEM((CHUNK, D), jnp.float32),
            pltpu.VMEM_SHARED((ACC_ROWS, D), jnp.float32),
            pltpu.SemaphoreType.DMA,
        ],
    )
    def _sc_edge(table_hbm, gidx_hbm, dst_hbm, zeros_hbm, out_hbm,
                 idx_v, dst_v, rows_v, acc, sem):
        c = lax.axis_index("c")
        s = lax.axis_index("s")
        wid = c * NSUB + s
        pltpu.sync_copy(gidx_hbm.at[wid], idx_v)
        pltpu.sync_copy(dst_hbm.at[wid], dst_v)
        pltpu.sync_copy(zeros_hbm, acc.at[pl.ds(s * ROWS_PT, ROWS_PT)])
        plsc.subcore_barrier()

        def chunk_body(g, carry):
            pltpu.async_copy(table_hbm.at[idx_v.at[g]], rows_v, sem).wait()
            pltpu.sync_copy(rows_v, acc.at[dst_v.at[g]], add=True)
            return carry

        lax.fori_loop(0, CPW, chunk_body, 0)
        plsc.subcore_barrier()

        @pl.when(s < NSUB - 1)
        def _():
            pltpu.sync_copy(acc.at[pl.ds(s * ROWS_PT, ROWS_PT)],
                            out_hbm.at[c, pl.ds(s * ROWS_PT, ROWS_PT)])

        @pl.when(s == NSUB - 1)
        def _():
            pltpu.sync_copy(acc.at[pl.ds((NSUB - 1) * ROWS_PT, LAST_ROWS)],
                            out_hbm.at[c, pl.ds((NSUB - 1) * ROWS_PT, LAST_ROWS)])

    return _sc_edge


RB = 1000


def _wh_body(h_ref, w_ref, b_ref, o_ref):
    o_ref[...] = (jnp.dot(h_ref[...], w_ref[...],
                          preferred_element_type=jnp.float32) + b_ref[...])


def _tc_wh(h, wcat, bcat):
    return pl.pallas_call(
        _wh_body,
        grid=(N // RB,),
        in_specs=[
            pl.BlockSpec((RB, D), lambda i: (i, 0)),
            pl.BlockSpec((D, T * D), lambda i: (0, 0)),
            pl.BlockSpec((1, T * D), lambda i: (0, 0)),
        ],
        out_specs=pl.BlockSpec((RB, T * D), lambda i: (i, 0)),
        out_shape=jax.ShapeDtypeStruct((N, T * D), jnp.float32),
    )(h, wcat, bcat)


def _gru_body(h_ref, a_ref, wih_ref, whh_ref, bih_ref, bhh_ref, o_ref):
    hb = h_ref[...]
    av = a_ref[...].astype(jnp.float32)
    ab = av[0] + av[1]
    gi = jnp.dot(ab, wih_ref[...], preferred_element_type=jnp.float32) + bih_ref[...]
    gh = jnp.dot(hb, whh_ref[...], preferred_element_type=jnp.float32) + bhh_ref[...]
    r = jax.nn.sigmoid(gi[:, :D] + gh[:, :D])
    z = jax.nn.sigmoid(gi[:, D:2 * D] + gh[:, D:2 * D])
    n = jnp.tanh(gi[:, 2 * D:] + r * gh[:, 2 * D:])
    o_ref[...] = (1.0 - z) * n + z * hb


def _tc_gru(h, a2, wiht, whht, bih, bhh):
    return pl.pallas_call(
        _gru_body,
        grid=(N // RB,),
        in_specs=[
            pl.BlockSpec((RB, D), lambda i: (i, 0)),
            pl.BlockSpec((NCORES, RB, D), lambda i: (0, i, 0)),
            pl.BlockSpec((D, 3 * D), lambda i: (0, 0)),
            pl.BlockSpec((D, 3 * D), lambda i: (0, 0)),
            pl.BlockSpec((1, 3 * D), lambda i: (0, 0)),
            pl.BlockSpec((1, 3 * D), lambda i: (0, 0)),
        ],
        out_specs=pl.BlockSpec((RB, D), lambda i: (i, 0)),
        out_shape=jax.ShapeDtypeStruct((N, D), jnp.float32),
    )(h, a2, wiht, whht, bih, bhh)


def _pool_body(h_ref, gid_ref, w1_ref, b1_ref, w2_ref, b2_ref, o_ref):
    hv = h_ref[...]
    gid = gid_ref[...]
    gids = lax.broadcasted_iota(jnp.int32, (1, NGRAPHS), 1)
    onehot = (gid == gids).astype(jnp.float32)
    sums = lax.dot_general(onehot, hv, (((0,), (0,)), ((), ())),
                           preferred_element_type=jnp.float32)
    cnt = jnp.sum(onehot, axis=0)[:, None]
    feat = sums / jnp.maximum(cnt, 1.0)
    hid = jax.nn.relu(jnp.dot(feat, w1_ref[...],
                              preferred_element_type=jnp.float32) + b1_ref[...])
    logits = jnp.dot(hid, w2_ref[...],
                     preferred_element_type=jnp.float32) + b2_ref[...]
    o_ref[...] = jax.nn.sigmoid(logits)


def _tc_pool(h, gid2d, w1, b1, w2, b2):
    return pl.pallas_call(
        _pool_body,
        out_shape=jax.ShapeDtypeStruct((NGRAPHS, 1), jnp.float32),
    )(h, gid2d, w1, b1, w2, b2)


def kernel(x, edge_index, edge_types, graph_ids, We, be, w_ih, w_hh,
           b_ih, b_hh, W1, b1, W2, b2):
    src = edge_index[0].astype(jnp.int32)
    dst = edge_index[1].astype(jnp.int32)
    et = edge_types.astype(jnp.int32)

    gidx = src * T + et
    pad = EPAD - E
    gidx_p = jnp.concatenate([gidx, jnp.zeros((pad,), jnp.int32)])
    dst_p = jnp.concatenate([dst, jnp.full((pad,), N, jnp.int32)])
    gidx_p = gidx_p.reshape(NWORKERS, CPW, CHUNK)
    dst_p = dst_p.reshape(NWORKERS, CPW, CHUNK)
    zeros = jnp.zeros((ROWS_PT, D), jnp.float32)

    wcat = jnp.transpose(We, (2, 0, 1)).reshape(D, T * D)
    bcat = be.reshape(1, T * D)
    wiht = w_ih.T
    whht = w_hh.T
    bih = b_ih.reshape(1, 3 * D)
    bhh = b_hh.reshape(1, 3 * D)

    h = x
    for _ in range(NSTEPS):
        wh = _tc_wh(h, wcat, bcat)
        table = wh.reshape(N * T, D)
        a2 = _make_sc_edge()(table, gidx_p, dst_p, zeros)
        h = _tc_gru(h, a2, wiht, whht, bih, bhh)

    gid2d = graph_ids.astype(jnp.int32).reshape(N, 1)
    return _tc_pool(h, gid2d, W1, b1.reshape(1, HID), W2, b2.reshape(1, 1))


# speedup vs baseline: 1.5221x; 1.0305x over previous
import functools

import jax
import jax.numpy as jnp
from jax import lax
from jax.experimental import pallas as pl
from jax.experimental.pallas import tpu as pltpu
from jax.experimental.pallas import tpu_sc as plsc

N = 10000
E = 320000
D = 128
T = 4
NSTEPS = 8
NGRAPHS = 16
HID = 256

NCORES = 2
NSUB = 16
NWORKERS = NCORES * NSUB
CHUNK = 128
CPW = 79
EPAD = NWORKERS * CPW * CHUNK
ROWS_PT = 632
ACC_ROWS = NSUB * ROWS_PT
LAST_ROWS = N - 15 * ROWS_PT


@functools.lru_cache(maxsize=None)
def _make_sc_edge():
    @functools.partial(
        pl.kernel,
        mesh=plsc.VectorSubcoreMesh(core_axis_name="c", subcore_axis_name="s"),
        out_type=jax.ShapeDtypeStruct((NCORES, N, D), jnp.float32),
        scratch_types=[
            pltpu.VMEM((CPW, CHUNK), jnp.int32),
            pltpu.VMEM((CPW, CHUNK), jnp.int32),
            pltpu.VMEM((CHUNK, D), jnp.float32),
            pltpu.VMEM_SHARED((ACC_ROWS, D), jnp.float32),
            pltpu.SemaphoreType.DMA,
        ],
    )
    def _sc_edge(table_hbm, gidx_hbm, dst_hbm, zeros_hbm, out_hbm,
                 idx_v, dst_v, rows_v, acc, sem):
        c = lax.axis_index("c")
        s = lax.axis_index("s")
        wid = c * NSUB + s
        pltpu.sync_copy(gidx_hbm.at[wid], idx_v)
        pltpu.sync_copy(dst_hbm.at[wid], dst_v)
        pltpu.sync_copy(zeros_hbm, acc.at[pl.ds(s * ROWS_PT, ROWS_PT)])
        plsc.subcore_barrier()

        def chunk_body(g, carry):
            pltpu.async_copy(table_hbm.at[idx_v.at[g]], rows_v, sem).wait()
            pltpu.sync_copy(rows_v, acc.at[dst_v.at[g]], add=True)
            return carry

        lax.fori_loop(0, CPW, chunk_body, 0)
        plsc.subcore_barrier()

        @pl.when(s < NSUB - 1)
        def _():
            pltpu.sync_copy(acc.at[pl.ds(s * ROWS_PT, ROWS_PT)],
                            out_hbm.at[c, pl.ds(s * ROWS_PT, ROWS_PT)])

        @pl.when(s == NSUB - 1)
        def _():
            pltpu.sync_copy(acc.at[pl.ds((NSUB - 1) * ROWS_PT, LAST_ROWS)],
                            out_hbm.at[c, pl.ds((NSUB - 1) * ROWS_PT, LAST_ROWS)])

    return _sc_edge


RB = 1000


def _wh_body(h_ref, w_ref, b_ref, o_ref):
    o_ref[...] = (jnp.dot(h_ref[...], w_ref[...],
                          preferred_element_type=jnp.float32) + b_ref[...])


def _tc_wh(h, wcat, bcat):
    return pl.pallas_call(
        _wh_body,
        grid=(N // RB,),
        in_specs=[
            pl.BlockSpec((RB, D), lambda i: (i, 0)),
            pl.BlockSpec((D, T * D), lambda i: (0, 0)),
            pl.BlockSpec((1, T * D), lambda i: (0, 0)),
        ],
        out_specs=pl.BlockSpec((RB, T * D), lambda i: (i, 0)),
        out_shape=jax.ShapeDtypeStruct((N, T * D), jnp.float32),
    )(h, wcat, bcat)


def _gru_core(h_ref, a_ref, wih_ref, whh_ref, bih_ref, bhh_ref):
    hb = h_ref[...]
    av = a_ref[...].astype(jnp.float32)
    ab = av[0] + av[1]
    gi = jnp.dot(ab, wih_ref[...], preferred_element_type=jnp.float32) + bih_ref[...]
    gh = jnp.dot(hb, whh_ref[...], preferred_element_type=jnp.float32) + bhh_ref[...]
    r = jax.nn.sigmoid(gi[:, :D] + gh[:, :D])
    z = jax.nn.sigmoid(gi[:, D:2 * D] + gh[:, D:2 * D])
    n = jnp.tanh(gi[:, 2 * D:] + r * gh[:, 2 * D:])
    return (1.0 - z) * n + z * hb


def _gru_body(h_ref, a_ref, wih_ref, whh_ref, bih_ref, bhh_ref, o_ref):
    o_ref[...] = _gru_core(h_ref, a_ref, wih_ref, whh_ref, bih_ref, bhh_ref)


def _tc_gru(h, a2, wiht, whht, bih, bhh):
    return pl.pallas_call(
        _gru_body,
        grid=(N // RB,),
        in_specs=[
            pl.BlockSpec((RB, D), lambda i: (i, 0)),
            pl.BlockSpec((NCORES, RB, D), lambda i: (0, i, 0)),
            pl.BlockSpec((D, 3 * D), lambda i: (0, 0)),
            pl.BlockSpec((D, 3 * D), lambda i: (0, 0)),
            pl.BlockSpec((1, 3 * D), lambda i: (0, 0)),
            pl.BlockSpec((1, 3 * D), lambda i: (0, 0)),
        ],
        out_specs=pl.BlockSpec((RB, D), lambda i: (i, 0)),
        out_shape=jax.ShapeDtypeStruct((N, D), jnp.float32),
    )(h, a2, wiht, whht, bih, bhh)


def _gruwh_body(h_ref, a_ref, wih_ref, whh_ref, bih_ref, bhh_ref,
                w_ref, b_ref, oh_ref, ow_ref):
    hn = _gru_core(h_ref, a_ref, wih_ref, whh_ref, bih_ref, bhh_ref)
    oh_ref[...] = hn
    ow_ref[...] = (jnp.dot(hn, w_ref[...],
                           preferred_element_type=jnp.float32) + b_ref[...])


def _tc_gruwh(h, a2, wiht, whht, bih, bhh, wcat, bcat):
    return pl.pallas_call(
        _gruwh_body,
        grid=(N // RB,),
        in_specs=[
            pl.BlockSpec((RB, D), lambda i: (i, 0)),
            pl.BlockSpec((NCORES, RB, D), lambda i: (0, i, 0)),
            pl.BlockSpec((D, 3 * D), lambda i: (0, 0)),
            pl.BlockSpec((D, 3 * D), lambda i: (0, 0)),
            pl.BlockSpec((1, 3 * D), lambda i: (0, 0)),
            pl.BlockSpec((1, 3 * D), lambda i: (0, 0)),
            pl.BlockSpec((D, T * D), lambda i: (0, 0)),
            pl.BlockSpec((1, T * D), lambda i: (0, 0)),
        ],
        out_specs=[
            pl.BlockSpec((RB, D), lambda i: (i, 0)),
            pl.BlockSpec((RB, T * D), lambda i: (i, 0)),
        ],
        out_shape=[
            jax.ShapeDtypeStruct((N, D), jnp.float32),
            jax.ShapeDtypeStruct((N, T * D), jnp.float32),
        ],
    )(h, a2, wiht, whht, bih, bhh, wcat, bcat)


def _pool_body(h_ref, gid_ref, w1_ref, b1_ref, w2_ref, b2_ref, o_ref):
    hv = h_ref[...]
    gid = gid_ref[...]
    gids = lax.broadcasted_iota(jnp.int32, (1, NGRAPHS), 1)
    onehot = (gid == gids).astype(jnp.float32)
    sums = lax.dot_general(onehot, hv, (((0,), (0,)), ((), ())),
                           preferred_element_type=jnp.float32)
    cnt = jnp.sum(onehot, axis=0)[:, None]
    feat = sums / jnp.maximum(cnt, 1.0)
    hid = jax.nn.relu(jnp.dot(feat, w1_ref[...],
                              preferred_element_type=jnp.float32) + b1_ref[...])
    logits = jnp.dot(hid, w2_ref[...],
                     preferred_element_type=jnp.float32) + b2_ref[...]
    o_ref[...] = jax.nn.sigmoid(logits)


def _tc_pool(h, gid2d, w1, b1, w2, b2):
    return pl.pallas_call(
        _pool_body,
        out_shape=jax.ShapeDtypeStruct((NGRAPHS, 1), jnp.float32),
    )(h, gid2d, w1, b1, w2, b2)


def kernel(x, edge_index, edge_types, graph_ids, We, be, w_ih, w_hh,
           b_ih, b_hh, W1, b1, W2, b2):
    src = edge_index[0].astype(jnp.int32)
    dst = edge_index[1].astype(jnp.int32)
    et = edge_types.astype(jnp.int32)

    gidx = src * T + et
    pad = EPAD - E
    gidx_p = jnp.concatenate([gidx, jnp.zeros((pad,), jnp.int32)])
    dst_p = jnp.concatenate([dst, jnp.full((pad,), N, jnp.int32)])
    gidx_p = gidx_p.reshape(NWORKERS, CPW, CHUNK)
    dst_p = dst_p.reshape(NWORKERS, CPW, CHUNK)
    zeros = jnp.zeros((ROWS_PT, D), jnp.float32)

    wcat = jnp.transpose(We, (2, 0, 1)).reshape(D, T * D)
    bcat = be.reshape(1, T * D)
    wiht = w_ih.T
    whht = w_hh.T
    bih = b_ih.reshape(1, 3 * D)
    bhh = b_hh.reshape(1, 3 * D)

    h = x
    wh = _tc_wh(h, wcat, bcat)
    for step in range(NSTEPS):
        table = wh.reshape(N * T, D)
        a2 = _make_sc_edge()(table, gidx_p, dst_p, zeros)
        if step < NSTEPS - 1:
            h, wh = _tc_gruwh(h, a2, wiht, whht, bih, bhh, wcat, bcat)
        else:
            h = _tc_gru(h, a2, wiht, whht, bih, bhh)

    gid2d = graph_ids.astype(jnp.int32).reshape(N, 1)
    return _tc_pool(h, gid2d, W1, b1.reshape(1, HID), W2, b2.reshape(1, 1))
